# baseline (speedup 1.0000x reference)
import functools

import jax
import jax.numpy as jnp
from jax import lax
from jax.experimental import pallas as pl
from jax.experimental.pallas import tpu as pltpu
from jax.experimental.pallas import tpu_sc as plsc

NUM_EXPERTS = 64
TOP_K = 8
HIDDEN = 4096
TOKENS = 32768
BLK_T = 512
NCHUNK = 8
CT = TOKENS // NCHUNK

_NUM_WORKERS = 32
_ROWS = CT // _NUM_WORKERS
_PAIRS = _ROWS // 2


def _logits_block(h_ref, w_ref, logits_ref):
    h = h_ref[...].astype(jnp.bfloat16)
    acc = jnp.dot(h, w_ref[...], preferred_element_type=jnp.float32)
    logits_ref[...] = acc.astype(jnp.bfloat16).astype(jnp.float32)


def _tc_logits_chunk(hidden, wt, c):
    nblk = CT // BLK_T
    return pl.pallas_call(
        _logits_block,
        grid=(nblk,),
        in_specs=[
            pl.BlockSpec((BLK_T, HIDDEN), lambda i, c=c: (c * nblk + i, 0)),
            pl.BlockSpec((HIDDEN, NUM_EXPERTS), lambda i: (0, 0)),
        ],
        out_specs=pl.BlockSpec((BLK_T, NUM_EXPERTS), lambda i: (i, 0)),
        out_shape=jax.ShapeDtypeStruct((CT, NUM_EXPERTS), jnp.float32),
        compiler_params=pltpu.CompilerParams(
            dimension_semantics=("arbitrary",),
        ),
    )(hidden, wt)


def _gather16(x, idx):
    return x.at[idx].get(mode="promise_in_bounds")


_SC_MESH = plsc.VectorSubcoreMesh(core_axis_name="c", subcore_axis_name="s")


def _make_sc_topk(cbase):
  @functools.partial(
      pl.kernel,
      mesh=_SC_MESH,
      compiler_params=pltpu.CompilerParams(needs_layout_passes=False,
                                           use_tc_tiling_on_sc=True),
      out_type=[],
      scratch_types=[
          pltpu.VMEM((_ROWS, NUM_EXPERTS), jnp.float32),
          pltpu.VMEM((_ROWS * TOP_K,), jnp.int32),
          pltpu.VMEM((_ROWS * TOP_K // 2,), jnp.int32),
      ],
  )
  def _sc_topk(logits_hbm, idx_ref, wts_ref, lg_ref, in_v, idx_v, wts_v):
    wid = lax.axis_index("s") * 2 + lax.axis_index("c")
    base = wid * _ROWS
    pltpu.sync_copy(logits_hbm.at[pl.ds(base, _ROWS)], in_v)
    obase = cbase + wid * _ROWS
    pltpu.sync_copy(in_v, lg_ref.at[pl.ds(obase, _ROWS)])

    lane = lax.iota(jnp.int32, 16)
    low8 = lane < 8
    lo_idx = lane & 7
    max_idx = jnp.where(low8, 0, 8)
    seven = jnp.full((16,), 7, jnp.int32)
    fifteen = jnp.full((16,), 15, jnp.int32)
    sign = jnp.uint32(0x80000000)
    himask = jnp.uint32(0xFFFF0000)

    def _sortd(k):
      return plsc.sort_key_val(k, lane, descending=True)[0]

    def token_top(tok):
      srt = []
      for j in range(4):
        v = in_v[tok, pl.ds(j * 16, 16)]
        bits = lax.bitcast_convert_type(v, jnp.uint32)
        mono = jnp.where((bits >> 31) == 1, ~bits, bits ^ sign)
        tie = (63 - (lane + 16 * j)).astype(jnp.uint32)
        srt.append(_sortd((mono & himask) | tie))

      def merge(a, b):
        return _sortd(jnp.where(low8, a, _gather16(b, lo_idx)))

      return merge(merge(srt[0], srt[1]), merge(srt[2], srt[3]))

    evens = 2 * lo_idx
    odds = evens + 1
    half = jnp.uint32(0x7FFF)

    def pair_weights(p):
      fa = token_top(2 * p)
      fb = token_top(2 * p + 1)
      pk = jnp.where(low8, fa, _gather16(fb, lo_idx))
      expert = 63 - lax.bitcast_convert_type(pk & jnp.uint32(63), jnp.int32)
      vbits = jnp.where((pk >> 31) == 1, (pk ^ sign) & himask,
                        (~pk) & himask)
      v = lax.bitcast_convert_type(vbits, jnp.float32)
      e = jnp.exp(v - _gather16(v, max_idx))
      cs = jnp.cumsum(e)
      s_a = _gather16(cs, seven)
      denom = jnp.where(low8, s_a, _gather16(cs, fifteen) - s_a)
      idx_v[pl.ds(16 * p, 16)] = expert
      w = e / denom
      b = lax.bitcast_convert_type(w, jnp.uint32)
      r = (b + half + ((b >> 16) & jnp.uint32(1))) >> 16
      return _gather16(r, evens) | (_gather16(r, odds) << 16)

    def body(q, carry):
      pa = pair_weights(2 * q)
      pb = pair_weights(2 * q + 1)
      packed = jnp.where(low8, pa, _gather16(pb, lo_idx))
      wts_v[pl.ds(16 * q, 16)] = lax.bitcast_convert_type(packed, jnp.int32)
      return carry

    lax.fori_loop(0, _PAIRS // 2, body, None)

    fbase = obase * TOP_K
    pltpu.sync_copy(idx_v, idx_ref.at[pl.ds(fbase, _ROWS * TOP_K)])
    pltpu.sync_copy(
        wts_v,
        wts_ref.at[pl.ds(pl.multiple_of(fbase // 2, 512),
                         _ROWS * TOP_K // 2)])

  return _sc_topk


_SC_KERNELS = [_make_sc_topk(c * CT) for c in range(NCHUNK)]


def kernel(hidden, W):
    wt = W.astype(jnp.bfloat16).T
    idx_buf = jax.new_ref(jnp.zeros((TOKENS * TOP_K,), jnp.int32))
    wts_buf = jax.new_ref(jnp.zeros((TOKENS * TOP_K // 2,), jnp.int32))
    lg_buf = jax.new_ref(jnp.zeros((TOKENS, NUM_EXPERTS), jnp.float32))
    for c in range(NCHUNK):
        lg = _tc_logits_chunk(hidden, wt, c)
        _SC_KERNELS[c](lg, idx_buf, wts_buf, lg_buf)
    indices = idx_buf[...].reshape(TOKENS, TOP_K)
    weights = lax.bitcast_convert_type(
        wts_buf[...], jnp.bfloat16).reshape(TOKENS, TOP_K)
    logits = lg_buf[...]
    return (indices, weights, logits)


# speedup vs baseline: 1.0760x; 1.0760x over previous
import functools

import jax
import jax.numpy as jnp
from jax import lax
from jax.experimental import pallas as pl
from jax.experimental.pallas import tpu as pltpu
from jax.experimental.pallas import tpu_sc as plsc

NUM_EXPERTS = 64
TOP_K = 8
HIDDEN = 4096
TOKENS = 32768
BLK_T = 512
NCHUNK = 8
CT = TOKENS // NCHUNK

_NUM_WORKERS = 32
_ROWS = CT // _NUM_WORKERS
_PAIRS = _ROWS // 2


def _logits_block(h_ref, w_ref, logits_ref):
    h = h_ref[...].astype(jnp.bfloat16)
    acc = jnp.dot(h, w_ref[...], preferred_element_type=jnp.float32)
    logits_ref[...] = acc.astype(jnp.bfloat16).astype(jnp.float32)


def _tc_logits_chunk(hidden, wt, c):
    nblk = CT // BLK_T
    return pl.pallas_call(
        _logits_block,
        grid=(nblk,),
        in_specs=[
            pl.BlockSpec((BLK_T, HIDDEN), lambda i, c=c: (c * nblk + i, 0)),
            pl.BlockSpec((HIDDEN, NUM_EXPERTS), lambda i: (0, 0)),
        ],
        out_specs=pl.BlockSpec((BLK_T, NUM_EXPERTS), lambda i: (i, 0)),
        out_shape=jax.ShapeDtypeStruct((CT, NUM_EXPERTS), jnp.float32),
        compiler_params=pltpu.CompilerParams(
            dimension_semantics=("arbitrary",),
        ),
    )(hidden, wt)


def _gather16(x, idx):
    return x.at[idx].get(mode="promise_in_bounds")


_SC_MESH = plsc.VectorSubcoreMesh(core_axis_name="c", subcore_axis_name="s")


def _make_sc_topk(cbase):
  @functools.partial(
      pl.kernel,
      mesh=_SC_MESH,
      compiler_params=pltpu.CompilerParams(needs_layout_passes=False,
                                           use_tc_tiling_on_sc=True),
      out_type=[],
      scratch_types=[
          pltpu.VMEM((_ROWS, NUM_EXPERTS), jnp.float32),
          pltpu.VMEM((_ROWS, TOP_K), jnp.int32),
          pltpu.VMEM((_ROWS, TOP_K), jnp.float32),
      ],
  )
  def _sc_topk(logits_hbm, idx_ref, wts_ref, lg_ref, in_v, idx_v, wts_v):
    wid = lax.axis_index("s") * 2 + lax.axis_index("c")
    base = wid * _ROWS
    pltpu.sync_copy(logits_hbm.at[pl.ds(base, _ROWS)], in_v)
    obase = cbase + wid * _ROWS
    pltpu.sync_copy(in_v, lg_ref.at[pl.ds(obase, _ROWS)])

    lane = lax.iota(jnp.int32, 16)
    low8 = lane < 8
    lo_idx = lane & 7
    max_idx = jnp.where(low8, 0, 8)
    seven = jnp.full((16,), 7, jnp.int32)
    fifteen = jnp.full((16,), 15, jnp.int32)
    sign = jnp.uint32(0x80000000)
    himask = jnp.uint32(0xFFFF0000)

    def _sortd(k):
      return plsc.sort_key_val(k, lane, descending=True)[0]

    def token_top(tok):
      srt = []
      for j in range(4):
        v = in_v[tok, pl.ds(j * 16, 16)]
        bits = lax.bitcast_convert_type(v, jnp.uint32)
        mono = jnp.where((bits >> 31) == 1, ~bits, bits ^ sign)
        tie = (63 - (lane + 16 * j)).astype(jnp.uint32)
        srt.append(_sortd((mono & himask) | tie))

      def merge(a, b):
        return _sortd(jnp.where(low8, a, _gather16(b, lo_idx)))

      return merge(merge(srt[0], srt[1]), merge(srt[2], srt[3]))

    def body(p, carry):
      fa = token_top(2 * p)
      fb = token_top(2 * p + 1)
      pk = jnp.where(low8, fa, _gather16(fb, lo_idx))
      expert = 63 - lax.bitcast_convert_type(pk & jnp.uint32(63), jnp.int32)
      vbits = jnp.where((pk >> 31) == 1, (pk ^ sign) & himask,
                        (~pk) & himask)
      v = lax.bitcast_convert_type(vbits, jnp.float32)
      e = jnp.exp(v - _gather16(v, max_idx))
      cs = jnp.cumsum(e)
      s_a = _gather16(cs, seven)
      denom = jnp.where(low8, s_a, _gather16(cs, fifteen) - s_a)
      rowv = jnp.where(low8, 2 * p, 2 * p + 1)
      plsc.store_scatter(idx_v, [rowv, lo_idx], expert)
      plsc.store_scatter(wts_v, [rowv, lo_idx], e / denom)
      return carry

    lax.fori_loop(0, _PAIRS, body, None)

    pltpu.sync_copy(idx_v, idx_ref.at[pl.ds(obase, _ROWS)])
    pltpu.sync_copy(wts_v, wts_ref.at[pl.ds(obase, _ROWS)])

  return _sc_topk


_SC_KERNELS = [_make_sc_topk(c * CT) for c in range(NCHUNK)]


def kernel(hidden, W):
    wt = W.astype(jnp.bfloat16).T
    idx_buf = jax.empty_ref(
        jax.ShapeDtypeStruct((TOKENS, TOP_K), jnp.int32))
    wts_buf = jax.empty_ref(
        jax.ShapeDtypeStruct((TOKENS, TOP_K), jnp.float32))
    lg_buf = jax.empty_ref(
        jax.ShapeDtypeStruct((TOKENS, NUM_EXPERTS), jnp.float32))
    for c in range(NCHUNK):
        lg = _tc_logits_chunk(hidden, wt, c)
        _SC_KERNELS[c](lg, idx_buf, wts_buf, lg_buf)
    indices = jax.freeze(idx_buf)
    weights = jax.freeze(wts_buf).astype(jnp.bfloat16)
    logits = jax.freeze(lg_buf)
    return (indices, weights, logits)


# speedup vs baseline: 1.1201x; 1.0410x over previous
import functools

import jax
import jax.numpy as jnp
from jax import lax
from jax.experimental import pallas as pl
from jax.experimental.pallas import tpu as pltpu
from jax.experimental.pallas import tpu_sc as plsc

NUM_EXPERTS = 64
TOP_K = 8
HIDDEN = 4096
TOKENS = 32768
BLK_T = 512
NCHUNK = 4
CT = TOKENS // NCHUNK

_NUM_WORKERS = 32
_ROWS = CT // _NUM_WORKERS
_PAIRS = _ROWS // 2


def _logits_block(h_ref, w_ref, logits_ref):
    h = h_ref[...].astype(jnp.bfloat16)
    acc = jnp.dot(h, w_ref[...], preferred_element_type=jnp.float32)
    logits_ref[...] = acc.astype(jnp.bfloat16).astype(jnp.float32)


def _tc_logits_chunk(hidden, wt, c):
    nblk = CT // BLK_T
    return pl.pallas_call(
        _logits_block,
        grid=(nblk,),
        in_specs=[
            pl.BlockSpec((BLK_T, HIDDEN), lambda i, c=c: (c * nblk + i, 0)),
            pl.BlockSpec((HIDDEN, NUM_EXPERTS), lambda i: (0, 0)),
        ],
        out_specs=pl.BlockSpec((BLK_T, NUM_EXPERTS), lambda i: (i, 0)),
        out_shape=jax.ShapeDtypeStruct((CT, NUM_EXPERTS), jnp.float32),
        compiler_params=pltpu.CompilerParams(
            dimension_semantics=("arbitrary",),
        ),
    )(hidden, wt)


def _gather16(x, idx):
    return x.at[idx].get(mode="promise_in_bounds")


_SC_MESH = plsc.VectorSubcoreMesh(core_axis_name="c", subcore_axis_name="s")


def _make_sc_topk(cbase):
  @functools.partial(
      pl.kernel,
      mesh=_SC_MESH,
      compiler_params=pltpu.CompilerParams(needs_layout_passes=False,
                                           use_tc_tiling_on_sc=True),
      out_type=[],
      scratch_types=[
          pltpu.VMEM((_ROWS, NUM_EXPERTS), jnp.float32),
          pltpu.VMEM((_ROWS, TOP_K), jnp.int32),
          pltpu.VMEM((_ROWS, TOP_K), jnp.float32),
      ],
  )
  def _sc_topk(logits_hbm, idx_ref, wts_ref, lg_ref, in_v, idx_v, wts_v):
    wid = lax.axis_index("s") * 2 + lax.axis_index("c")
    base = wid * _ROWS
    pltpu.sync_copy(logits_hbm.at[pl.ds(base, _ROWS)], in_v)
    obase = cbase + wid * _ROWS
    pltpu.sync_copy(in_v, lg_ref.at[pl.ds(obase, _ROWS)])

    lane = lax.iota(jnp.int32, 16)
    low8 = lane < 8
    lo_idx = lane & 7
    max_idx = jnp.where(low8, 0, 8)
    seven = jnp.full((16,), 7, jnp.int32)
    fifteen = jnp.full((16,), 15, jnp.int32)
    sign = jnp.uint32(0x80000000)
    himask = jnp.uint32(0xFFFF0000)

    def _sortd(k):
      return plsc.sort_key_val(k, lane, descending=True)[0]

    def token_top(tok):
      srt = []
      for j in range(4):
        v = in_v[tok, pl.ds(j * 16, 16)]
        bits = lax.bitcast_convert_type(v, jnp.uint32)
        mono = jnp.where((bits >> 31) == 1, ~bits, bits ^ sign)
        tie = (63 - (lane + 16 * j)).astype(jnp.uint32)
        srt.append(_sortd((mono & himask) | tie))

      def merge(a, b):
        return _sortd(jnp.where(low8, a, _gather16(b, lo_idx)))

      return merge(merge(srt[0], srt[1]), merge(srt[2], srt[3]))

    def body(p, carry):
      fa = token_top(2 * p)
      fb = token_top(2 * p + 1)
      pk = jnp.where(low8, fa, _gather16(fb, lo_idx))
      expert = 63 - lax.bitcast_convert_type(pk & jnp.uint32(63), jnp.int32)
      vbits = jnp.where((pk >> 31) == 1, (pk ^ sign) & himask,
                        (~pk) & himask)
      v = lax.bitcast_convert_type(vbits, jnp.float32)
      e = jnp.exp(v - _gather16(v, max_idx))
      cs = jnp.cumsum(e)
      s_a = _gather16(cs, seven)
      denom = jnp.where(low8, s_a, _gather16(cs, fifteen) - s_a)
      rowv = jnp.where(low8, 2 * p, 2 * p + 1)
      plsc.store_scatter(idx_v, [rowv, lo_idx], expert)
      plsc.store_scatter(wts_v, [rowv, lo_idx], e / denom)
      return carry

    lax.fori_loop(0, _PAIRS, body, None)

    pltpu.sync_copy(idx_v, idx_ref.at[pl.ds(obase, _ROWS)])
    pltpu.sync_copy(wts_v, wts_ref.at[pl.ds(obase, _ROWS)])

  return _sc_topk


_SC_KERNELS = [_make_sc_topk(c * CT) for c in range(NCHUNK)]


def kernel(hidden, W):
    wt = W.astype(jnp.bfloat16).T
    idx_buf = jax.empty_ref(
        jax.ShapeDtypeStruct((TOKENS, TOP_K), jnp.int32))
    wts_buf = jax.empty_ref(
        jax.ShapeDtypeStruct((TOKENS, TOP_K), jnp.float32))
    lg_buf = jax.empty_ref(
        jax.ShapeDtypeStruct((TOKENS, NUM_EXPERTS), jnp.float32))
    for c in range(NCHUNK):
        lg = _tc_logits_chunk(hidden, wt, c)
        _SC_KERNELS[c](lg, idx_buf, wts_buf, lg_buf)
    indices = jax.freeze(idx_buf)
    weights = jax.freeze(wts_buf).astype(jnp.bfloat16)
    logits = jax.freeze(lg_buf)
    return (indices, weights, logits)
